# baseline (speedup 1.0000x reference)
import jax
import jax.numpy as jnp
from jax import lax
from jax.experimental import pallas as pl
from jax.experimental.pallas import tpu as pltpu, tpu_sc as plsc

D = 1024
B, S = 4, 2048
N = B * S
NC, NS = 2, 16
NW = NC * NS
SPW = S // NW
PCH = 8
NSUB = SPW // PCH
ROWS = B * PCH
NBUF = 3
LANES = 16
VPR = D // LANES
SKEW = 6


def _emb_body(ids_hbm, tok_hbm, pos_hbm, out_hbm,
              idx_v, pos_v, rows0, rows1, rows2,
              sem_i, sem_p0, sem_p1, sem_g0, sem_g1, sem_g2,
              sem_o0, sem_o1, sem_o2):
    wid = lax.axis_index("s") * NC + lax.axis_index("c")
    s_base = wid * SPW

    rows = (rows0, rows1, rows2)
    sem_g = (sem_g0, sem_g1, sem_g2)
    sem_o = (sem_o0, sem_o1, sem_o2)
    sem_p = (sem_p0, sem_p1)

    id_cp = pltpu.make_async_copy(
        ids_hbm.at[pl.ds(wid * B * SPW, B * SPW)], idx_v, sem_i)
    id_cp.start()

    def pos_cp(s):
        return pltpu.make_async_copy(
            pos_hbm.at[pl.ds(s_base + s * PCH, PCH)],
            pos_v.at[pl.ds((s % 2) * PCH, PCH)], sem_p[s % 2])

    def gather_cp(s):
        q = s % NBUF
        return pltpu.make_async_copy(
            tok_hbm.at[idx_v.at[pl.ds(s * ROWS, ROWS)]], rows[q], sem_g[q])

    def out_cp(s, b):
        q = s % NBUF
        return pltpu.make_async_copy(
            rows[q].at[pl.ds(b * PCH, PCH)],
            out_hbm.at[pl.ds(b * S + s_base + s * PCH, PCH)], sem_o[q])

    id_cp.wait()
    gather_cp(0).start()
    gather_cp(1).start()
    pos_cp(0).start()
    pos_cp(1).start()

    for s in range(NSUB):
        q = s % NBUF
        gather_cp(s).wait()
        pos_cp(s).wait()
        h = (s % 2) * PCH

        def row_body(r, carry):
            vals = {}
            for c in range(SKEW):
                vals[c] = pos_v[h + r, pl.ds(c * LANES, LANES)]
            for c in range(VPR):
                if c + SKEW < VPR:
                    vals[c + SKEW] = pos_v[h + r,
                                           pl.ds((c + SKEW) * LANES, LANES)]
                sl = pl.ds(c * LANES, LANES)
                pv = vals.pop(c)
                for b in range(B):
                    plsc.addupdate(rows[q].at[b * PCH + r, sl], pv)
            return carry

        lax.fori_loop(0, PCH, row_body, 0)
        for b in range(B):
            out_cp(s, b).start()
        if s + 2 < NSUB:
            pos_cp(s + 2).start()
            if s >= 1:
                for b in range(B):
                    out_cp(s - 1, b).wait()
            gather_cp(s + 2).start()
    for s in (NSUB - 3, NSUB - 2, NSUB - 1):
        for b in range(B):
            out_cp(s, b).wait()


def kernel(input_ids, token_table, pos_table):
    ids_flat = (input_ids.astype(jnp.int32)
                .reshape(B, NW, NSUB, PCH)
                .transpose(1, 2, 0, 3)
                .reshape(N))
    mesh = plsc.VectorSubcoreMesh(core_axis_name="c", subcore_axis_name="s")
    out = pl.kernel(
        _emb_body,
        out_type=jax.ShapeDtypeStruct((N, D), jnp.float32),
        mesh=mesh,
        scratch_types=(
            [pltpu.VMEM((B * SPW,), jnp.int32),
             pltpu.VMEM((2 * PCH, D), jnp.float32)]
            + [pltpu.VMEM((ROWS, D), jnp.float32) for _ in range(NBUF)]
            + [pltpu.SemaphoreType.DMA for _ in range(3 + 2 * NBUF)]
        ),
    )(ids_flat, token_table, pos_table)
    return out.reshape(B, S, D)


# speedup vs baseline: 1.0109x; 1.0109x over previous
import jax
import jax.numpy as jnp
from jax import lax
from jax.experimental import pallas as pl
from jax.experimental.pallas import tpu as pltpu, tpu_sc as plsc

D = 1024
B, S = 4, 2048
N = B * S
NC, NS = 2, 16
NW = NC * NS
SPW = S // NW
PCH = 8
NSUB = SPW // PCH
ROWS = B * PCH
NBUF = 3
LANES = 16
VPR = D // LANES
SKEW = 6


def _emb_body(ids_hbm, tok_hbm, pos_hbm, out_hbm,
              idx_v, pos_v, rows0, rows1, rows2,
              sem_i, sem_p0, sem_p1, sem_g0, sem_g1, sem_g2,
              sem_o0, sem_o1, sem_o2):
    wid = lax.axis_index("s") * NC + lax.axis_index("c")
    s_base = wid * SPW

    rows = (rows0, rows1, rows2)
    sem_g = (sem_g0, sem_g1, sem_g2)
    sem_o = (sem_o0, sem_o1, sem_o2)
    sem_p = (sem_p0, sem_p1)

    id_cp = pltpu.make_async_copy(
        ids_hbm.at[pl.ds(wid * B * SPW, B * SPW)], idx_v, sem_i)
    id_cp.start()

    def pos_cp(s):
        return pltpu.make_async_copy(
            pos_hbm.at[pl.ds(s_base + s * PCH, PCH)],
            pos_v.at[pl.ds((s % 2) * PCH, PCH)], sem_p[s % 2])

    pos_cp(0).start()
    pos_cp(1).start()

    def gather_cp(s):
        q = s % NBUF
        return pltpu.make_async_copy(
            tok_hbm.at[idx_v.at[pl.ds(s * ROWS, ROWS)]], rows[q], sem_g[q])

    def out_cp(s, b):
        q = s % NBUF
        return pltpu.make_async_copy(
            rows[q].at[pl.ds(b * PCH, PCH)],
            out_hbm.at[pl.ds(b * S + s_base + s * PCH, PCH)], sem_o[q])

    id_cp.wait()
    gather_cp(0).start()
    gather_cp(1).start()

    for s in range(NSUB):
        q = s % NBUF
        gather_cp(s).wait()
        pos_cp(s).wait()
        h = (s % 2) * PCH

        def row_body(r, carry):
            for c in range(VPR):
                sl = pl.ds(c * LANES, LANES)
                pv = pos_v[h + r, sl]
                for b in range(B):
                    plsc.addupdate(rows[q].at[b * PCH + r, sl], pv)
            return carry

        lax.fori_loop(0, PCH, row_body, 0)
        for b in range(B):
            out_cp(s, b).start()
        if s + 2 < NSUB:
            pos_cp(s + 2).start()
            if s >= 1:
                for b in range(B):
                    out_cp(s - 1, b).wait()
            gather_cp(s + 2).start()
    for s in (NSUB - 3, NSUB - 2, NSUB - 1):
        for b in range(B):
            out_cp(s, b).wait()


def kernel(input_ids, token_table, pos_table):
    ids_flat = (input_ids.astype(jnp.int32)
                .reshape(B, NW, NSUB, PCH)
                .transpose(1, 2, 0, 3)
                .reshape(N))
    mesh = plsc.VectorSubcoreMesh(core_axis_name="c", subcore_axis_name="s")
    out = pl.kernel(
        _emb_body,
        out_type=jax.ShapeDtypeStruct((N, D), jnp.float32),
        mesh=mesh,
        scratch_types=(
            [pltpu.VMEM((B * SPW,), jnp.int32),
             pltpu.VMEM((2 * PCH, D), jnp.float32)]
            + [pltpu.VMEM((ROWS, D), jnp.float32) for _ in range(NBUF)]
            + [pltpu.SemaphoreType.DMA for _ in range(3 + 2 * NBUF)]
        ),
    )(ids_flat, token_table, pos_table)
    return out.reshape(B, S, D)
